# baseline (speedup 1.0000x reference)
import functools

import jax
import jax.numpy as jnp
from jax import lax
from jax.experimental import pallas as pl
from jax.experimental.pallas import tpu as pltpu
from jax.experimental.pallas import tpu_sc as plsc

T, D, H, E, A = 4096, 768, 64, 14, 128
EP = 16
TILE_T = 1024
K = 3
NC, NS = 2, 16
NW = NC * NS
RPW = T // NW


def _gating_kernel(x_ref, wg1_ref, bg1_ref, wg2p_ref, bg2p_ref, w_ref):
    h = jnp.maximum(
        jnp.dot(x_ref[...], wg1_ref[...], preferred_element_type=jnp.float32)
        + bg1_ref[...], 0.0)
    logits_t = lax.dot_general(
        wg2p_ref[...], h, (((0,), (1,)), ((), ())),
        preferred_element_type=jnp.float32) + bg2p_ref[...].T
    m = jnp.max(logits_t, axis=0, keepdims=True)
    ex = jnp.exp(logits_t - m)
    w_ref[...] = (ex / jnp.sum(ex, axis=0, keepdims=True)).T


def _expert_kernel(x_ref, cb_ref, wer_ref, be_ref, y_ref):
    x = x_ref[...]
    combine = cb_ref[...][:, :E]
    acc = jnp.dot(combine, be_ref[...], preferred_element_type=jnp.float32)
    for e in range(E):
        xe = jnp.dot(x, wer_ref[e], preferred_element_type=jnp.float32)
        acc = acc + combine[:, e:e + 1] * xe
    y_ref[...] = acc


def _sc_topk(w_hbm, out_hbm, w_v, cb_v):
    wid = lax.axis_index("s") * NC + lax.axis_index("c")
    base = wid * (RPW * EP)
    pltpu.sync_copy(w_hbm.at[pl.ds(base, RPW * EP)], w_v)

    def body(t, carry):
        w = w_v[pl.ds(t * EP, EP)]
        iota = lax.broadcasted_iota(jnp.int32, (EP,), 0)
        skey, sval = plsc.sort_key_val(w, iota, descending=True)
        cb_v[pl.ds(t * EP, EP)] = jnp.zeros((EP,), jnp.float32)
        plsc.store_scatter(cb_v, [t * EP + sval], skey, mask=iota < K)
        return carry

    lax.fori_loop(0, RPW, body, 0)
    pltpu.sync_copy(cb_v, out_hbm.at[pl.ds(base, RPW * EP)])


@functools.partial(jax.jit, static_argnames=())
def _run(x, Wg1, bg1, Wg2p, bg2p, We, be):
    weights = pl.pallas_call(
        _gating_kernel,
        grid=(T // TILE_T,),
        in_specs=[
            pl.BlockSpec((TILE_T, D), lambda i: (i, 0)),
            pl.BlockSpec((D, H), lambda i: (0, 0)),
            pl.BlockSpec((1, H), lambda i: (0, 0)),
            pl.BlockSpec((H, EP), lambda i: (0, 0)),
            pl.BlockSpec((1, EP), lambda i: (0, 0)),
        ],
        out_specs=pl.BlockSpec((TILE_T, EP), lambda i: (i, 0)),
        out_shape=jax.ShapeDtypeStruct((T, EP), jnp.float32),
    )(x, Wg1, bg1, Wg2p, bg2p)

    mesh = plsc.VectorSubcoreMesh(core_axis_name="c", subcore_axis_name="s")
    comb_flat = pl.kernel(
        _sc_topk,
        mesh=mesh,
        out_type=jax.ShapeDtypeStruct((T * EP,), jnp.float32),
        scratch_types=[
            pltpu.VMEM((RPW * EP,), jnp.float32),
            pltpu.VMEM((RPW * EP,), jnp.float32),
        ],
        compiler_params=pltpu.CompilerParams(needs_layout_passes=False),
    )(weights.reshape(T * EP))

    return pl.pallas_call(
        _expert_kernel,
        grid=(T // TILE_T,),
        in_specs=[
            pl.BlockSpec((TILE_T, D), lambda i: (i, 0)),
            pl.BlockSpec((TILE_T, EP), lambda i: (i, 0)),
            pl.BlockSpec((E, D, A), lambda i: (0, 0, 0)),
            pl.BlockSpec((E, A), lambda i: (0, 0)),
        ],
        out_specs=pl.BlockSpec((TILE_T, A), lambda i: (i, 0)),
        out_shape=jax.ShapeDtypeStruct((T, A), jnp.float32),
    )(x, comb_flat.reshape(T, EP), We, be)


def kernel(x, Wg1, bg1, Wg2, bg2, We, be, top_k):
    Wg2p = jnp.pad(Wg2, ((0, 0), (0, EP - E)))
    bg2p = jnp.concatenate(
        [bg2, jnp.full((EP - E,), -1e30, jnp.float32)]).reshape(1, EP)
    return _run(x, Wg1, bg1.reshape(1, H), Wg2p, bg2p, We, be)


# speedup vs baseline: 1.8775x; 1.8775x over previous
import functools

import jax
import jax.numpy as jnp
from jax import lax
from jax.experimental import pallas as pl

T, D, H, E, A = 4096, 768, 64, 14, 128
TILE_T = 1024
K = 3


def _moe_kernel(x_ref, wg1_ref, bg1_ref, wg2_ref, bg2_ref, wer_ref, be_ref,
                y_ref):
    x = x_ref[...]
    h = jnp.maximum(
        jnp.dot(x, wg1_ref[...], preferred_element_type=jnp.float32)
        + bg1_ref[...], 0.0)
    logits_t = lax.dot_general(
        wg2_ref[...], h, (((0,), (1,)), ((), ())),
        preferred_element_type=jnp.float32) + bg2_ref[...].T
    m = jnp.max(logits_t, axis=0, keepdims=True)
    ex = jnp.exp(logits_t - m)
    w = ex / jnp.sum(ex, axis=0, keepdims=True)

    row = lax.broadcasted_iota(jnp.int32, w.shape, 0)
    mask = jnp.zeros(w.shape, jnp.bool_)
    for _ in range(K):
        cand = jnp.where(mask, -1.0, w)
        mx = jnp.max(cand, axis=0, keepdims=True)
        first = jnp.min(jnp.where(cand == mx, row, E), axis=0, keepdims=True)
        mask = mask | (row == first)
    combine = jnp.where(mask, w, 0.0).T

    acc0 = jnp.dot(combine, be_ref[...], preferred_element_type=jnp.float32)
    acc1 = jnp.zeros_like(acc0)
    for e in range(E):
        xe = jnp.dot(x, wer_ref[e], preferred_element_type=jnp.float32)
        if e % 2 == 0:
            acc0 = acc0 + combine[:, e:e + 1] * xe
        else:
            acc1 = acc1 + combine[:, e:e + 1] * xe
    y_ref[...] = acc0 + acc1


@functools.partial(jax.jit, static_argnames=())
def _run(x, Wg1, bg1, Wg2, bg2, We, be):
    grid = (T // TILE_T,)
    return pl.pallas_call(
        _moe_kernel,
        grid=grid,
        in_specs=[
            pl.BlockSpec((TILE_T, D), lambda i: (i, 0)),
            pl.BlockSpec((D, H), lambda i: (0, 0)),
            pl.BlockSpec((1, H), lambda i: (0, 0)),
            pl.BlockSpec((H, E), lambda i: (0, 0)),
            pl.BlockSpec((1, E), lambda i: (0, 0)),
            pl.BlockSpec((E, D, A), lambda i: (0, 0, 0)),
            pl.BlockSpec((E, A), lambda i: (0, 0)),
        ],
        out_specs=pl.BlockSpec((TILE_T, A), lambda i: (i, 0)),
        out_shape=jax.ShapeDtypeStruct((T, A), jnp.float32),
    )(x, Wg1, bg1, Wg2, bg2, We, be)


def kernel(x, Wg1, bg1, Wg2, bg2, We, be, top_k):
    return _run(x, Wg1, bg1.reshape(1, H), Wg2, bg2.reshape(1, E), We, be)
